# baseline (speedup 1.0000x reference)
import functools

import jax
import jax.numpy as jnp
from jax import lax
from jax.experimental import pallas as pl
from jax.experimental.pallas import tpu as pltpu
from jax.experimental.pallas import tpu_sc as plsc

_NC = 2
_NS = 16
_NW = _NC * _NS

_CH = 128
_NBUF = 2
_BLK = 400


def _sc_mesh():
    return plsc.VectorSubcoreMesh(core_axis_name="c", subcore_axis_name="s")


def _build_deg(E, NPAD):
    EPW = E // _NW
    full = EPW // _CH
    tail = EPW - full * _CH
    RPT = NPAD // _NS
    M = full // _NBUF
    assert full == M * _NBUF and M >= 2, (full, _NBUF)

    scratch = (
        [
            pltpu.VMEM_SHARED((NPAD,), jnp.float32),
            pltpu.VMEM((RPT,), jnp.float32),
            pltpu.VMEM((_CH,), jnp.float32),
        ]
        + [pltpu.VMEM((_CH,), jnp.int32) for _ in range(_NBUF)]
        + [pltpu.SemaphoreType.DMA for _ in range(2 * _NBUF)]
        + ([pltpu.VMEM((tail,), jnp.int32)] if tail else [])
    )

    @functools.partial(
        pl.kernel,
        mesh=_sc_mesh(),
        out_type=jax.ShapeDtypeStruct((_NC, NPAD), jnp.float32),
        scratch_types=scratch,
    )
    def deg_kernel(row_hbm, degp, dacc, zbuf, ones, *rest):
        ir = rest[0:_NBUF]
        isem = rest[_NBUF:2 * _NBUF]
        ssem = rest[2 * _NBUF:3 * _NBUF]
        if tail:
            irt = rest[3 * _NBUF]

        cid = lax.axis_index("c")
        sid = lax.axis_index("s")
        wid = cid * _NS + sid
        base = wid * EPW
        zsl = pl.ds(sid * RPT, RPT)

        zero16 = jnp.zeros((16,), jnp.float32)
        one16 = jnp.ones((16,), jnp.float32)

        def fill(i, _):
            zbuf[pl.ds(i * 16, 16)] = zero16
            return 0

        lax.fori_loop(0, RPT // 16, fill, 0)
        for j in range(_CH // 16):
            ones[pl.ds(j * 16, 16)] = one16
        pltpu.sync_copy(zbuf, dacc.at[zsl])
        plsc.subcore_barrier()

        def idx_dma(c, b):
            off = base + c * _CH
            pltpu.async_copy(row_hbm.at[pl.ds(off, _CH)], ir[b], isem[b])

        def idx_wait(b):
            pltpu.make_async_copy(
                row_hbm.at[pl.ds(0, _CH)], ir[b], isem[b]).wait()

        def sct_wait(b):
            pltpu.make_async_copy(
                row_hbm.at[pl.ds(0, _CH)], ir[b], ssem[b]).wait()

        for b in range(_NBUF):
            idx_dma(b, b)

        def group(m, _):
            for b in range(_NBUF):
                idx_wait(b)
                pltpu.async_copy(ones, dacc.at[ir[b]], ssem[b], add=True)
            for b in range(_NBUF):
                sct_wait(b)
                idx_dma(m * _NBUF + b + _NBUF, b)
            return 0

        lax.fori_loop(0, M - 1, group, 0)
        for b in range(_NBUF):
            idx_wait(b)
            pltpu.async_copy(ones, dacc.at[ir[b]], ssem[b], add=True)
        for b in range(_NBUF):
            sct_wait(b)
        if tail:
            pltpu.sync_copy(row_hbm.at[pl.ds(base + full * _CH, tail)], irt)
            pltpu.sync_copy(ones.at[pl.ds(0, tail)], dacc.at[irt], add=True)

        plsc.subcore_barrier()
        pltpu.sync_copy(dacc.at[zsl], degp.at[cid, zsl])

    return deg_kernel


def _build_edges(E, NPAD, D):
    EPW = E // _NW
    full = EPW // _CH
    tail = EPW - full * _CH
    RPT = NPAD // _NS
    NZ = RPT // _CH
    M = full // _NBUF
    assert full == M * _NBUF and M >= 2, (full, _NBUF)

    scratch = (
        [
            pltpu.VMEM_SHARED((NPAD, D), jnp.float32),
            pltpu.VMEM_SHARED((NPAD,), jnp.float32),
            pltpu.VMEM((EPW,), jnp.int32),
            pltpu.VMEM((RPT,), jnp.float32),
            pltpu.VMEM((_CH,), jnp.float32),
        ]
        + [pltpu.VMEM((_CH, D), jnp.float32) for _ in range(_NBUF)]
        + [pltpu.VMEM((_CH,), jnp.int32) for _ in range(_NBUF)]
        + [pltpu.SemaphoreType.DMA for _ in range(2 * _NBUF)]
        + ([pltpu.VMEM((tail, D), jnp.float32), pltpu.VMEM((tail,), jnp.int32)]
           if tail else [])
        + [pltpu.SemaphoreType.DMA, pltpu.SemaphoreType.DMA]
    )

    @functools.partial(
        pl.kernel,
        mesh=_sc_mesh(),
        out_type=(
            jax.ShapeDtypeStruct((_NC, NPAD, D), jnp.float32),
            jax.ShapeDtypeStruct((_NC, NPAD), jnp.float32),
        ),
        scratch_types=scratch,
    )
    def edges(g_hbm, row_hbm, col_hbm, p_hbm, cntp, acc, cacc, rowall, zbuf,
              ones, *rest):
        rows = rest[0:_NBUF]
        ic = rest[_NBUF:2 * _NBUF]
        gsem = rest[2 * _NBUF:3 * _NBUF]
        csem = rest[3 * _NBUF:4 * _NBUF]
        if tail:
            rows_t, ict = rest[4 * _NBUF:4 * _NBUF + 2]
        zsem = rest[-2]
        tsem = rest[-1]

        cid = lax.axis_index("c")
        sid = lax.axis_index("s")
        wid = cid * _NS + sid
        base = wid * EPW

        zero16 = jnp.zeros((16,), jnp.float32)
        one16 = jnp.ones((16,), jnp.float32)

        def zrow(i, _):
            for j in range(D // 16):
                rows[0][i, pl.ds(j * 16, 16)] = zero16
            return 0

        lax.fori_loop(0, _CH, zrow, 0)

        def fill(i, _):
            zbuf[pl.ds(i * 16, 16)] = zero16
            return 0

        lax.fori_loop(0, RPT // 16, fill, 0)
        for j in range(_CH // 16):
            ones[pl.ds(j * 16, 16)] = one16

        for z in range(NZ):
            pltpu.sync_copy(rows[0], acc.at[pl.ds(sid * RPT + z * _CH, _CH)])
        pltpu.sync_copy(zbuf, cacc.at[pl.ds(sid * RPT, RPT)])
        pltpu.sync_copy(row_hbm.at[pl.ds(base, EPW)], rowall)
        plsc.subcore_barrier()

        def issue(c, b):
            pltpu.async_copy(col_hbm.at[pl.ds(base + c * _CH, _CH)], ic[b],
                             csem[b])
            pltpu.async_copy(g_hbm.at[rowall.at[pl.ds(c * _CH, _CH)]], rows[b],
                             gsem[b])

        def wait(b):
            pltpu.make_async_copy(
                col_hbm.at[pl.ds(0, _CH)], ic[b], csem[b]).wait()
            pltpu.make_async_copy(
                g_hbm.at[pl.ds(0, _CH)], rows[b], gsem[b]).wait()

        for b in range(_NBUF):
            issue(b, b)

        def cnt_wait(b):
            pltpu.make_async_copy(
                col_hbm.at[pl.ds(0, _CH)], ic[b], tsem).wait()

        def group(m, _):
            for b in range(_NBUF):
                c = m * _NBUF + b
                wait(b)
                pltpu.async_copy(ones, cacc.at[ic[b]], tsem, add=True)
                pltpu.sync_copy(rows[b], acc.at[ic[b]], add=True)
                cnt_wait(b)
                issue(c + _NBUF, b)
            return 0

        lax.fori_loop(0, M - 1, group, 0)
        for b in range(_NBUF):
            wait(b)
            pltpu.async_copy(ones, cacc.at[ic[b]], tsem, add=True)
            pltpu.sync_copy(rows[b], acc.at[ic[b]], add=True)
            cnt_wait(b)
        if tail:
            off = base + full * _CH
            pltpu.sync_copy(col_hbm.at[pl.ds(off, tail)], ict)
            pltpu.async_copy(
                g_hbm.at[rowall.at[pl.ds(full * _CH, tail)]], rows_t,
                gsem[0]).wait()
            pltpu.sync_copy(ones.at[pl.ds(0, tail)], cacc.at[ict], add=True)
            pltpu.sync_copy(rows_t, acc.at[ict], add=True)

        plsc.subcore_barrier()
        osl = pl.ds(sid * RPT, RPT)
        pltpu.sync_copy(acc.at[osl], p_hbm.at[cid, osl])
        pltpu.sync_copy(cacc.at[osl], cntp.at[cid, osl])

    return edges


def _matmul_body(x_ref, w_ref, b_ref, h_ref):
    h_ref[...] = jnp.dot(x_ref[...], w_ref[...],
                         preferred_element_type=jnp.float32) + b_ref[...]


def _scale_body(h_ref, dg_ref, g_ref):
    d = dg_ref[...]
    deg = d[0] + d[1] + 1.0
    g_ref[...] = h_ref[...] * lax.rsqrt(deg)


def _combine_body(p_ref, g_ref, dg_ref, cn_ref, o_ref):
    p = p_ref[...]
    d = dg_ref[...]
    c = cn_ref[...]
    deg = d[0] + d[1] + 1.0
    cnt = c[0] + c[1] + 1.0
    o_ref[...] = (p[0] + p[1] + g_ref[...]) * (lax.rsqrt(deg) / cnt)


def _tc_matmul(x, W, b2):
    N, D_IN = x.shape
    D_OUT = W.shape[1]
    return pl.pallas_call(
        _matmul_body,
        grid=(N // _BLK,),
        in_specs=[
            pl.BlockSpec((_BLK, D_IN), lambda i: (i, 0)),
            pl.BlockSpec((D_IN, D_OUT), lambda i: (0, 0)),
            pl.BlockSpec((1, D_OUT), lambda i: (0, 0)),
        ],
        out_specs=pl.BlockSpec((_BLK, D_OUT), lambda i: (i, 0)),
        out_shape=jax.ShapeDtypeStruct((N, D_OUT), jnp.float32),
    )(x, W, b2)


def _tc_scale(h, degr):
    N, D_OUT = h.shape
    return pl.pallas_call(
        _scale_body,
        grid=(N // _BLK,),
        in_specs=[
            pl.BlockSpec((_BLK, D_OUT), lambda i: (i, 0)),
            pl.BlockSpec((_NC, _BLK, 1), lambda i: (0, i, 0)),
        ],
        out_specs=pl.BlockSpec((_BLK, D_OUT), lambda i: (i, 0)),
        out_shape=jax.ShapeDtypeStruct((N, D_OUT), jnp.float32),
    )(h, degr)


def _tc_combine(P, g, degr, cntr):
    N, D_OUT = g.shape
    return pl.pallas_call(
        _combine_body,
        grid=(N // _BLK,),
        in_specs=[
            pl.BlockSpec((_NC, _BLK, D_OUT), lambda i: (0, i, 0)),
            pl.BlockSpec((_BLK, D_OUT), lambda i: (i, 0)),
            pl.BlockSpec((_NC, _BLK, 1), lambda i: (0, i, 0)),
            pl.BlockSpec((_NC, _BLK, 1), lambda i: (0, i, 0)),
        ],
        out_specs=pl.BlockSpec((_BLK, D_OUT), lambda i: (i, 0)),
        out_shape=jax.ShapeDtypeStruct((N, D_OUT), jnp.float32),
    )(P, g, degr, cntr)


def kernel(x, edge_index, W, b):
    N, D_IN = x.shape
    D_OUT = W.shape[1]
    E = edge_index.shape[1]
    assert E % _NW == 0 and N % _BLK == 0, (E, N)
    group = _NS * _CH
    NPAD = -(-N // group) * group

    row = edge_index[0]
    col = edge_index[1]

    degp = _build_deg(E, NPAD)(row)
    degr = degp.reshape(_NC, NPAD, 1)

    h = _tc_matmul(x, W, b.reshape(1, D_OUT))
    g = _tc_scale(h, degr)

    P, cntp = _build_edges(E, NPAD, D_OUT)(g, row, col)
    cntr = cntp.reshape(_NC, NPAD, 1)

    out = _tc_combine(P, g, degr, cntr)
    return out


# speedup vs baseline: 1.0187x; 1.0187x over previous
import functools

import jax
import jax.numpy as jnp
from jax import lax
from jax.experimental import pallas as pl
from jax.experimental.pallas import tpu as pltpu
from jax.experimental.pallas import tpu_sc as plsc

_NC = 2
_NS = 16
_NW = _NC * _NS

_CH = 128
_NBUF = 2
_BLK = 400


def _sc_mesh():
    return plsc.VectorSubcoreMesh(core_axis_name="c", subcore_axis_name="s")


def _build_counts(E, NPAD):
    EPW = E // _NW
    full = EPW // _CH
    tail = EPW - full * _CH
    RPT = NPAD // _NS
    M = full // _NBUF
    assert full == M * _NBUF and M >= 2, (full, _NBUF)

    scratch = (
        [
            pltpu.VMEM_SHARED((NPAD,), jnp.float32),
            pltpu.VMEM_SHARED((NPAD,), jnp.float32),
            pltpu.VMEM((RPT,), jnp.float32),
            pltpu.VMEM((_CH,), jnp.float32),
        ]
        + [pltpu.VMEM((_CH,), jnp.int32) for _ in range(2 * _NBUF)]
        + [pltpu.SemaphoreType.DMA for _ in range(3 * _NBUF)]
        + ([pltpu.VMEM((tail,), jnp.int32) for _ in range(2)] if tail else [])
    )

    @functools.partial(
        pl.kernel,
        mesh=_sc_mesh(),
        out_type=(
            jax.ShapeDtypeStruct((_NC, NPAD), jnp.float32),
            jax.ShapeDtypeStruct((_NC, NPAD), jnp.float32),
        ),
        scratch_types=scratch,
    )
    def counts(row_hbm, col_hbm, degp, cntp, dacc, cacc, zbuf, ones, *rest):
        ir = rest[0:_NBUF]
        ic = rest[_NBUF:2 * _NBUF]
        isem = rest[2 * _NBUF:4 * _NBUF]
        ssem = rest[4 * _NBUF:5 * _NBUF]
        if tail:
            irt, ict = rest[5 * _NBUF:5 * _NBUF + 2]

        cid = lax.axis_index("c")
        sid = lax.axis_index("s")
        wid = cid * _NS + sid
        base = wid * EPW
        zsl = pl.ds(sid * RPT, RPT)

        zero16 = jnp.zeros((16,), jnp.float32)
        one16 = jnp.ones((16,), jnp.float32)

        def fill(i, _):
            zbuf[pl.ds(i * 16, 16)] = zero16
            return 0

        lax.fori_loop(0, RPT // 16, fill, 0)
        for j in range(_CH // 16):
            ones[pl.ds(j * 16, 16)] = one16
        pltpu.sync_copy(zbuf, dacc.at[zsl])
        pltpu.sync_copy(zbuf, cacc.at[zsl])
        plsc.subcore_barrier()

        def idx_dma(c, b):
            off = base + c * _CH
            pltpu.async_copy(row_hbm.at[pl.ds(off, _CH)], ir[b], isem[2 * b])
            pltpu.async_copy(col_hbm.at[pl.ds(off, _CH)], ic[b], isem[2 * b + 1])

        def idx_wait(b):
            pltpu.make_async_copy(
                row_hbm.at[pl.ds(0, _CH)], ir[b], isem[2 * b]).wait()
            pltpu.make_async_copy(
                col_hbm.at[pl.ds(0, _CH)], ic[b], isem[2 * b + 1]).wait()

        def sct_wait(b):
            pltpu.make_async_copy(
                row_hbm.at[pl.ds(0, _CH)], ir[b], ssem[b]).wait()
            pltpu.make_async_copy(
                col_hbm.at[pl.ds(0, _CH)], ic[b], ssem[b]).wait()

        for b in range(_NBUF):
            idx_dma(b, b)

        def group(m, _):
            for b in range(_NBUF):
                idx_wait(b)
                pltpu.async_copy(ones, dacc.at[ir[b]], ssem[b], add=True)
                pltpu.async_copy(ones, cacc.at[ic[b]], ssem[b], add=True)
            for b in range(_NBUF):
                sct_wait(b)
                idx_dma(m * _NBUF + b + _NBUF, b)
            return 0

        lax.fori_loop(0, M - 1, group, 0)
        for b in range(_NBUF):
            idx_wait(b)
            pltpu.async_copy(ones, dacc.at[ir[b]], ssem[b], add=True)
            pltpu.async_copy(ones, cacc.at[ic[b]], ssem[b], add=True)
        for b in range(_NBUF):
            sct_wait(b)
        if tail:
            off = base + full * _CH
            pltpu.sync_copy(row_hbm.at[pl.ds(off, tail)], irt)
            pltpu.sync_copy(col_hbm.at[pl.ds(off, tail)], ict)
            pltpu.sync_copy(ones.at[pl.ds(0, tail)], dacc.at[irt], add=True)
            pltpu.sync_copy(ones.at[pl.ds(0, tail)], cacc.at[ict], add=True)

        plsc.subcore_barrier()
        pltpu.sync_copy(dacc.at[zsl], degp.at[cid, zsl])
        pltpu.sync_copy(cacc.at[zsl], cntp.at[cid, zsl])

    return counts


def _build_edges(E, NPAD, D):
    EPW = E // _NW
    full = EPW // _CH
    tail = EPW - full * _CH
    RPT = NPAD // _NS
    NZ = RPT // _CH
    M = full // _NBUF
    assert full == M * _NBUF and M >= 2, (full, _NBUF)

    scratch = (
        [
            pltpu.VMEM_SHARED((NPAD, D), jnp.float32),
            pltpu.VMEM((EPW,), jnp.int32),
        ]
        + [pltpu.VMEM((_CH, D), jnp.float32) for _ in range(_NBUF)]
        + [pltpu.VMEM((_CH,), jnp.int32) for _ in range(_NBUF)]
        + [pltpu.SemaphoreType.DMA for _ in range(2 * _NBUF)]
        + ([pltpu.VMEM((tail, D), jnp.float32), pltpu.VMEM((tail,), jnp.int32)]
           if tail else [])
    )

    @functools.partial(
        pl.kernel,
        mesh=_sc_mesh(),
        out_type=jax.ShapeDtypeStruct((_NC, NPAD, D), jnp.float32),
        scratch_types=scratch,
    )
    def edges(g_hbm, row_hbm, col_hbm, p_hbm, acc, rowall, *rest):
        rows = rest[0:_NBUF]
        ic = rest[_NBUF:2 * _NBUF]
        gsem = rest[2 * _NBUF:3 * _NBUF]
        csem = rest[3 * _NBUF:4 * _NBUF]
        if tail:
            rows_t, ict = rest[4 * _NBUF:4 * _NBUF + 2]

        cid = lax.axis_index("c")
        sid = lax.axis_index("s")
        wid = cid * _NS + sid
        base = wid * EPW

        zero16 = jnp.zeros((16,), jnp.float32)

        def zrow(i, _):
            for j in range(D // 16):
                rows[0][i, pl.ds(j * 16, 16)] = zero16
            return 0

        lax.fori_loop(0, _CH, zrow, 0)
        for z in range(NZ):
            pltpu.sync_copy(rows[0], acc.at[pl.ds(sid * RPT + z * _CH, _CH)])
        pltpu.sync_copy(row_hbm.at[pl.ds(base, EPW)], rowall)
        plsc.subcore_barrier()

        def issue(c, b):
            pltpu.async_copy(col_hbm.at[pl.ds(base + c * _CH, _CH)], ic[b],
                             csem[b])
            pltpu.async_copy(g_hbm.at[rowall.at[pl.ds(c * _CH, _CH)]], rows[b],
                             gsem[b])

        def wait(b):
            pltpu.make_async_copy(
                col_hbm.at[pl.ds(0, _CH)], ic[b], csem[b]).wait()
            pltpu.make_async_copy(
                g_hbm.at[pl.ds(0, _CH)], rows[b], gsem[b]).wait()

        for b in range(_NBUF):
            issue(b, b)

        def group(m, _):
            for b in range(_NBUF):
                c = m * _NBUF + b
                wait(b)
                pltpu.sync_copy(rows[b], acc.at[ic[b]], add=True)
                issue(c + _NBUF, b)
            return 0

        lax.fori_loop(0, M - 1, group, 0)
        for b in range(_NBUF):
            wait(b)
            pltpu.sync_copy(rows[b], acc.at[ic[b]], add=True)
        if tail:
            off = base + full * _CH
            pltpu.sync_copy(col_hbm.at[pl.ds(off, tail)], ict)
            pltpu.async_copy(
                g_hbm.at[rowall.at[pl.ds(full * _CH, tail)]], rows_t,
                gsem[0]).wait()
            pltpu.sync_copy(rows_t, acc.at[ict], add=True)

        plsc.subcore_barrier()
        osl = pl.ds(sid * RPT, RPT)
        pltpu.sync_copy(acc.at[osl], p_hbm.at[cid, osl])

    return edges


def _matmul_body(x_ref, w_ref, b_ref, h_ref):
    h_ref[...] = jnp.dot(x_ref[...], w_ref[...],
                         preferred_element_type=jnp.float32) + b_ref[...]


def _scale_body(h_ref, dg_ref, g_ref):
    d = dg_ref[...]
    deg = d[0] + d[1] + 1.0
    g_ref[...] = h_ref[...] * lax.rsqrt(deg)


def _combine_body(p_ref, g_ref, dg_ref, cn_ref, o_ref):
    p = p_ref[...]
    d = dg_ref[...]
    c = cn_ref[...]
    deg = d[0] + d[1] + 1.0
    cnt = c[0] + c[1] + 1.0
    o_ref[...] = (p[0] + p[1] + g_ref[...]) * (lax.rsqrt(deg) / cnt)


def _tc_matmul(x, W, b2):
    N, D_IN = x.shape
    D_OUT = W.shape[1]
    return pl.pallas_call(
        _matmul_body,
        grid=(N // _BLK,),
        in_specs=[
            pl.BlockSpec((_BLK, D_IN), lambda i: (i, 0)),
            pl.BlockSpec((D_IN, D_OUT), lambda i: (0, 0)),
            pl.BlockSpec((1, D_OUT), lambda i: (0, 0)),
        ],
        out_specs=pl.BlockSpec((_BLK, D_OUT), lambda i: (i, 0)),
        out_shape=jax.ShapeDtypeStruct((N, D_OUT), jnp.float32),
    )(x, W, b2)


def _tc_scale(h, degr):
    N, D_OUT = h.shape
    return pl.pallas_call(
        _scale_body,
        grid=(N // _BLK,),
        in_specs=[
            pl.BlockSpec((_BLK, D_OUT), lambda i: (i, 0)),
            pl.BlockSpec((_NC, _BLK, 1), lambda i: (0, i, 0)),
        ],
        out_specs=pl.BlockSpec((_BLK, D_OUT), lambda i: (i, 0)),
        out_shape=jax.ShapeDtypeStruct((N, D_OUT), jnp.float32),
    )(h, degr)


def _tc_combine(P, g, degr, cntr):
    N, D_OUT = g.shape
    return pl.pallas_call(
        _combine_body,
        grid=(N // _BLK,),
        in_specs=[
            pl.BlockSpec((_NC, _BLK, D_OUT), lambda i: (0, i, 0)),
            pl.BlockSpec((_BLK, D_OUT), lambda i: (i, 0)),
            pl.BlockSpec((_NC, _BLK, 1), lambda i: (0, i, 0)),
            pl.BlockSpec((_NC, _BLK, 1), lambda i: (0, i, 0)),
        ],
        out_specs=pl.BlockSpec((_BLK, D_OUT), lambda i: (i, 0)),
        out_shape=jax.ShapeDtypeStruct((N, D_OUT), jnp.float32),
    )(P, g, degr, cntr)


def kernel(x, edge_index, W, b):
    N, D_IN = x.shape
    D_OUT = W.shape[1]
    E = edge_index.shape[1]
    assert E % _NW == 0 and N % _BLK == 0, (E, N)
    group = _NS * _CH
    NPAD = -(-N // group) * group

    row = edge_index[0]
    col = edge_index[1]

    degp, cntp = _build_counts(E, NPAD)(row, col)
    degr = degp.reshape(_NC, NPAD, 1)
    cntr = cntp.reshape(_NC, NPAD, 1)

    h = _tc_matmul(x, W, b.reshape(1, D_OUT))
    g = _tc_scale(h, degr)

    P = _build_edges(E, NPAD, D_OUT)(g, row, col)

    out = _tc_combine(P, g, degr, cntr)
    return out


# speedup vs baseline: 1.1165x; 1.0960x over previous
import functools

import jax
import jax.numpy as jnp
from jax import lax
from jax.experimental import pallas as pl
from jax.experimental.pallas import tpu as pltpu
from jax.experimental.pallas import tpu_sc as plsc

_NC = 2
_NS = 16
_NW = _NC * _NS

_CH = 128
_NBUF = 2
_BLK = 2000


def _sc_mesh():
    return plsc.VectorSubcoreMesh(core_axis_name="c", subcore_axis_name="s")


def _build_counts(E, NPAD):
    EPW = E // _NW
    full = EPW // _CH
    tail = EPW - full * _CH
    RPT = NPAD // _NS
    M = full // _NBUF
    assert full == M * _NBUF and M >= 2, (full, _NBUF)

    scratch = (
        [
            pltpu.VMEM_SHARED((NPAD,), jnp.float32),
            pltpu.VMEM_SHARED((NPAD,), jnp.float32),
            pltpu.VMEM((RPT,), jnp.float32),
            pltpu.VMEM((_CH,), jnp.float32),
        ]
        + [pltpu.VMEM((_CH,), jnp.int32) for _ in range(2 * _NBUF)]
        + [pltpu.SemaphoreType.DMA for _ in range(3 * _NBUF)]
        + ([pltpu.VMEM((tail,), jnp.int32) for _ in range(2)] if tail else [])
    )

    @functools.partial(
        pl.kernel,
        mesh=_sc_mesh(),
        out_type=(
            jax.ShapeDtypeStruct((_NC, NPAD), jnp.float32),
            jax.ShapeDtypeStruct((_NC, NPAD), jnp.float32),
        ),
        scratch_types=scratch,
    )
    def counts(row_hbm, col_hbm, degp, cntp, dacc, cacc, zbuf, ones, *rest):
        ir = rest[0:_NBUF]
        ic = rest[_NBUF:2 * _NBUF]
        isem = rest[2 * _NBUF:4 * _NBUF]
        ssem = rest[4 * _NBUF:5 * _NBUF]
        if tail:
            irt, ict = rest[5 * _NBUF:5 * _NBUF + 2]

        cid = lax.axis_index("c")
        sid = lax.axis_index("s")
        wid = cid * _NS + sid
        base = wid * EPW
        zsl = pl.ds(sid * RPT, RPT)

        zero16 = jnp.zeros((16,), jnp.float32)
        one16 = jnp.ones((16,), jnp.float32)

        def fill(i, _):
            zbuf[pl.ds(i * 16, 16)] = zero16
            return 0

        lax.fori_loop(0, RPT // 16, fill, 0)
        for j in range(_CH // 16):
            ones[pl.ds(j * 16, 16)] = one16
        pltpu.sync_copy(zbuf, dacc.at[zsl])
        pltpu.sync_copy(zbuf, cacc.at[zsl])
        plsc.subcore_barrier()

        def idx_dma(c, b):
            off = base + c * _CH
            pltpu.async_copy(row_hbm.at[pl.ds(off, _CH)], ir[b], isem[2 * b])
            pltpu.async_copy(col_hbm.at[pl.ds(off, _CH)], ic[b], isem[2 * b + 1])

        def idx_wait(b):
            pltpu.make_async_copy(
                row_hbm.at[pl.ds(0, _CH)], ir[b], isem[2 * b]).wait()
            pltpu.make_async_copy(
                col_hbm.at[pl.ds(0, _CH)], ic[b], isem[2 * b + 1]).wait()

        def sct_wait(b):
            pltpu.make_async_copy(
                row_hbm.at[pl.ds(0, _CH)], ir[b], ssem[b]).wait()
            pltpu.make_async_copy(
                col_hbm.at[pl.ds(0, _CH)], ic[b], ssem[b]).wait()

        for b in range(_NBUF):
            idx_dma(b, b)

        def group(m, _):
            for b in range(_NBUF):
                idx_wait(b)
                pltpu.async_copy(ones, dacc.at[ir[b]], ssem[b], add=True)
                pltpu.async_copy(ones, cacc.at[ic[b]], ssem[b], add=True)
            for b in range(_NBUF):
                sct_wait(b)
                idx_dma(m * _NBUF + b + _NBUF, b)
            return 0

        lax.fori_loop(0, M - 1, group, 0)
        for b in range(_NBUF):
            idx_wait(b)
            pltpu.async_copy(ones, dacc.at[ir[b]], ssem[b], add=True)
            pltpu.async_copy(ones, cacc.at[ic[b]], ssem[b], add=True)
        for b in range(_NBUF):
            sct_wait(b)
        if tail:
            off = base + full * _CH
            pltpu.sync_copy(row_hbm.at[pl.ds(off, tail)], irt)
            pltpu.sync_copy(col_hbm.at[pl.ds(off, tail)], ict)
            pltpu.sync_copy(ones.at[pl.ds(0, tail)], dacc.at[irt], add=True)
            pltpu.sync_copy(ones.at[pl.ds(0, tail)], cacc.at[ict], add=True)

        plsc.subcore_barrier()
        pltpu.sync_copy(dacc.at[zsl], degp.at[cid, zsl])
        pltpu.sync_copy(cacc.at[zsl], cntp.at[cid, zsl])

    return counts


def _build_edges(E, NPAD, D):
    EPW = E // _NW
    full = EPW // _CH
    tail = EPW - full * _CH
    RPT = NPAD // _NS
    NZ = RPT // _CH
    M = full // _NBUF
    assert full == M * _NBUF and M >= 2, (full, _NBUF)

    scratch = (
        [
            pltpu.VMEM_SHARED((NPAD, D), jnp.float32),
            pltpu.VMEM((EPW,), jnp.int32),
        ]
        + [pltpu.VMEM((_CH, D), jnp.float32) for _ in range(_NBUF)]
        + [pltpu.VMEM((_CH,), jnp.int32) for _ in range(_NBUF)]
        + [pltpu.SemaphoreType.DMA for _ in range(2 * _NBUF)]
        + ([pltpu.VMEM((tail, D), jnp.float32), pltpu.VMEM((tail,), jnp.int32)]
           if tail else [])
    )

    @functools.partial(
        pl.kernel,
        mesh=_sc_mesh(),
        out_type=jax.ShapeDtypeStruct((_NC, NPAD, D), jnp.float32),
        scratch_types=scratch,
    )
    def edges(g_hbm, row_hbm, col_hbm, p_hbm, acc, rowall, *rest):
        rows = rest[0:_NBUF]
        ic = rest[_NBUF:2 * _NBUF]
        gsem = rest[2 * _NBUF:3 * _NBUF]
        csem = rest[3 * _NBUF:4 * _NBUF]
        if tail:
            rows_t, ict = rest[4 * _NBUF:4 * _NBUF + 2]

        cid = lax.axis_index("c")
        sid = lax.axis_index("s")
        wid = cid * _NS + sid
        base = wid * EPW

        zero16 = jnp.zeros((16,), jnp.float32)

        def zrow(i, _):
            for j in range(D // 16):
                rows[0][i, pl.ds(j * 16, 16)] = zero16
            return 0

        lax.fori_loop(0, _CH, zrow, 0)
        for z in range(NZ):
            pltpu.sync_copy(rows[0], acc.at[pl.ds(sid * RPT + z * _CH, _CH)])
        pltpu.sync_copy(row_hbm.at[pl.ds(base, EPW)], rowall)
        plsc.subcore_barrier()

        def issue(c, b):
            pltpu.async_copy(col_hbm.at[pl.ds(base + c * _CH, _CH)], ic[b],
                             csem[b])
            pltpu.async_copy(g_hbm.at[rowall.at[pl.ds(c * _CH, _CH)]], rows[b],
                             gsem[b])

        def wait(b):
            pltpu.make_async_copy(
                col_hbm.at[pl.ds(0, _CH)], ic[b], csem[b]).wait()
            pltpu.make_async_copy(
                g_hbm.at[pl.ds(0, _CH)], rows[b], gsem[b]).wait()

        for b in range(_NBUF):
            issue(b, b)

        def group(m, _):
            for b in range(_NBUF):
                c = m * _NBUF + b
                wait(b)
                pltpu.sync_copy(rows[b], acc.at[ic[b]], add=True)
                issue(c + _NBUF, b)
            return 0

        lax.fori_loop(0, M - 1, group, 0)
        for b in range(_NBUF):
            wait(b)
            pltpu.sync_copy(rows[b], acc.at[ic[b]], add=True)
        if tail:
            off = base + full * _CH
            pltpu.sync_copy(col_hbm.at[pl.ds(off, tail)], ict)
            pltpu.async_copy(
                g_hbm.at[rowall.at[pl.ds(full * _CH, tail)]], rows_t,
                gsem[0]).wait()
            pltpu.sync_copy(rows_t, acc.at[ict], add=True)

        plsc.subcore_barrier()
        osl = pl.ds(sid * RPT, RPT)
        pltpu.sync_copy(acc.at[osl], p_hbm.at[cid, osl])

    return edges


def _linear_body(x_ref, w_ref, b_ref, dg_ref, g_ref):
    h = jnp.dot(x_ref[...], w_ref[...], preferred_element_type=jnp.float32)
    h = h + b_ref[...]
    d = dg_ref[...]
    deg = d[0] + d[1] + 1.0
    g_ref[...] = h * lax.rsqrt(deg)


def _combine_body(p_ref, g_ref, dg_ref, cn_ref, o_ref):
    p = p_ref[...]
    d = dg_ref[...]
    c = cn_ref[...]
    deg = d[0] + d[1] + 1.0
    cnt = c[0] + c[1] + 1.0
    o_ref[...] = (p[0] + p[1] + g_ref[...]) * (lax.rsqrt(deg) / cnt)


def _tc_linear(x, W, b2, degr):
    N, D_IN = x.shape
    D_OUT = W.shape[1]
    return pl.pallas_call(
        _linear_body,
        grid=(N // _BLK,),
        in_specs=[
            pl.BlockSpec((_BLK, D_IN), lambda i: (i, 0)),
            pl.BlockSpec((D_IN, D_OUT), lambda i: (0, 0)),
            pl.BlockSpec((1, D_OUT), lambda i: (0, 0)),
            pl.BlockSpec((_NC, _BLK, 1), lambda i: (0, i, 0)),
        ],
        out_specs=pl.BlockSpec((_BLK, D_OUT), lambda i: (i, 0)),
        out_shape=jax.ShapeDtypeStruct((N, D_OUT), jnp.float32),
    )(x, W, b2, degr)


def _tc_combine(P, g, degr, cntr):
    N, D_OUT = g.shape
    return pl.pallas_call(
        _combine_body,
        grid=(N // _BLK,),
        in_specs=[
            pl.BlockSpec((_NC, _BLK, D_OUT), lambda i: (0, i, 0)),
            pl.BlockSpec((_BLK, D_OUT), lambda i: (i, 0)),
            pl.BlockSpec((_NC, _BLK, 1), lambda i: (0, i, 0)),
            pl.BlockSpec((_NC, _BLK, 1), lambda i: (0, i, 0)),
        ],
        out_specs=pl.BlockSpec((_BLK, D_OUT), lambda i: (i, 0)),
        out_shape=jax.ShapeDtypeStruct((N, D_OUT), jnp.float32),
    )(P, g, degr, cntr)


def kernel(x, edge_index, W, b):
    N, D_IN = x.shape
    D_OUT = W.shape[1]
    E = edge_index.shape[1]
    assert E % _NW == 0 and N % _BLK == 0, (E, N)
    group = _NS * _CH
    NPAD = -(-N // group) * group

    row = edge_index[0]
    col = edge_index[1]

    degp, cntp = _build_counts(E, NPAD)(row, col)
    degr = degp.reshape(_NC, NPAD, 1)
    cntr = cntp.reshape(_NC, NPAD, 1)

    g = _tc_linear(x, W, b.reshape(1, D_OUT), degr)

    P = _build_edges(E, NPAD, D_OUT)(g, row, col)

    out = _tc_combine(P, g, degr, cntr)
    return out
